# baseline (speedup 1.0000x reference)
import functools

import jax
import jax.numpy as jnp
from jax import lax
from jax.experimental import pallas as pl
from jax.experimental.pallas import tpu as pltpu
from jax.experimental.pallas import tpu_sc as plsc

N = 10000
E = 160000
B = 64
NP = 10240
EP = 163840
NTILES = 32
EPT = EP // NTILES
K1 = 64
K2 = 128
STR = NP // 16
D1 = 112
D2 = 32
RB = 1024
GRID = NP // RB

f32 = jnp.float32


def _mm_body(x_ref, w_ref, o_ref):
    y = jnp.dot(x_ref[...], w_ref[...], preferred_element_type=f32)
    lane = lax.broadcasted_iota(jnp.int32, y.shape, 1)
    o_ref[...] = jnp.where(lane == 100, 1.0, y)


def _tc_y1(xp, w1t):
    return pl.pallas_call(
        _mm_body,
        grid=(GRID,),
        in_specs=[
            pl.BlockSpec((RB, 256), lambda i: (i, 0)),
            pl.BlockSpec((256, D1), lambda i: (0, 0)),
        ],
        out_specs=pl.BlockSpec((RB, D1), lambda i: (i, 0)),
        out_shape=jax.ShapeDtypeStruct((NP, D1), f32),
    )(xp, w1t)


@functools.lru_cache(maxsize=None)
def _make_sc_agg(D, KD, GRP, NC=2):
    NW = 16 * NC
    EPTc = EP // NW
    NCH = EPTc // KD
    NBODY = NCH // (2 * GRP)
    mesh = plsc.VectorSubcoreMesh(
        core_axis_name="c", subcore_axis_name="s", num_cores=NC, num_subcores=16
    )

    @functools.partial(
        pl.kernel,
        out_type=jax.ShapeDtypeStruct((NC * NP, D), f32),
        mesh=mesh,
        compiler_params=pltpu.CompilerParams(use_tc_tiling_on_sc=False),
        scratch_types=[
            pltpu.VMEM((EPTc,), jnp.int32),
            pltpu.VMEM((NCH, KD), jnp.int32),
            pltpu.VMEM((2 * GRP, KD, D), f32),
            pltpu.VMEM_SHARED((NP, D), f32),
            pltpu.SemaphoreType.DMA,
            pltpu.SemaphoreType.DMA,
            pltpu.SemaphoreType.DMA,
            pltpu.SemaphoreType.DMA,
        ],
    )
    def sc_agg(y_hbm, src_hbm, dst_hbm, z_hbm, out_hbm,
               src_all, dst_all, rows, agg, gsa, gsb, ssa, ssb):
        cid = lax.axis_index("c")
        sid = lax.axis_index("s")
        wid = sid * NC + cid

        def fire_g(c0, b0, sem):
            for b in range(GRP):
                idx = src_all.at[pl.ds((c0 + b) * KD, KD)]
                pltpu.async_copy(y_hbm.at[idx], rows.at[b0 + b], sem)

        def drain_g(c0, b0, sem):
            for b in range(GRP):
                idx = src_all.at[pl.ds((c0 + b) * KD, KD)]
                pltpu.make_async_copy(y_hbm.at[idx], rows.at[b0 + b], sem).wait()

        def fire_s(c0, b0, sem):
            for b in range(GRP):
                pltpu.async_copy(rows.at[b0 + b], agg.at[dst_all.at[c0 + b]],
                                 sem, add=True)

        def drain_s(c0, b0, sem):
            for b in range(GRP):
                pltpu.make_async_copy(rows.at[b0 + b],
                                      agg.at[dst_all.at[c0 + b]], sem).wait()

        pltpu.sync_copy(src_hbm.at[wid], src_all)
        pltpu.sync_copy(dst_hbm.at[wid], dst_all)
        pltpu.sync_copy(z_hbm, agg.at[pl.ds(sid * STR, STR)])
        plsc.subcore_barrier()

        fire_g(0, 0, gsa)

        def body(g, carry):
            cA = g * 2 * GRP
            cB = cA + GRP

            @pl.when(g > 0)
            def _():
                drain_s(cA - GRP, GRP, ssb)

            fire_g(cB, GRP, gsb)
            drain_g(cA, 0, gsa)
            fire_s(cA, 0, ssa)
            drain_g(cB, GRP, gsb)
            fire_s(cB, GRP, ssb)
            drain_s(cA, 0, ssa)

            @pl.when(g < NBODY - 1)
            def _():
                fire_g(cA + 2 * GRP, 0, gsa)

            return carry

        lax.fori_loop(0, NBODY, body, 0)
        drain_s((NBODY - 1) * 2 * GRP + GRP, GRP, ssb)
        plsc.subcore_barrier()
        row0 = cid * NP + sid * STR
        pltpu.sync_copy(agg.at[pl.ds(sid * STR, STR)], out_hbm.at[pl.ds(row0, STR)])

    return sc_agg


def _mid_body(y1_ref, a0_ref, w_ref, b_ref, o_ref):
    a = a0_ref[...]
    deg = a[:, 100:101]
    mean = a / jnp.maximum(deg, 1.0)
    h1 = jnp.maximum(jnp.where(deg > 0, mean, y1_ref[...]) + b_ref[...], 0.0)
    y2 = jnp.dot(h1, w_ref[...], preferred_element_type=f32)
    lane = lax.broadcasted_iota(jnp.int32, y2.shape, 1)
    o_ref[...] = jnp.where(lane == 20, 1.0, jnp.where(lane == 21, deg, y2))


def _tc_mid(y1, a0, w2t, b1p):
    return pl.pallas_call(
        _mid_body,
        grid=(GRID,),
        in_specs=[
            pl.BlockSpec((RB, D1), lambda i: (i, 0)),
            pl.BlockSpec((RB, D1), lambda i: (i, 0)),
            pl.BlockSpec((D1, D2), lambda i: (0, 0)),
            pl.BlockSpec((1, D1), lambda i: (0, 0)),
        ],
        out_specs=pl.BlockSpec((RB, D2), lambda i: (i, 0)),
        out_shape=jax.ShapeDtypeStruct((NP, D2), f32),
    )(y1, a0, w2t, b1p)


def _head_body(y2_ref, a0_ref, gid_ref, b2_ref, d2_ref,
               gwh_ref, gwd_ref, gb_ref, wv_ref, wc_ref, fb1_ref,
               bn1g_ref, bn1b_ref, f2t_ref, f2b_ref, bn2g_ref, bn2b_ref,
               f3_ref, f3b_ref, o_ref, acc_ref):
    i = pl.program_id(0)

    @pl.when(i == 0)
    def _():
        acc_ref[...] = jnp.zeros((B, D2), f32)

    a = a0_ref[...]
    y2 = y2_ref[...]
    deg = y2[:, 21:22]
    mean = a / jnp.maximum(deg, 1.0)
    h2 = jnp.maximum(jnp.where(deg > 0, mean, y2) + b2_ref[...], 0.0)
    lane = lax.broadcasted_iota(jnp.int32, h2.shape, 1)
    h2 = jnp.where(lane > 20, 0.0, h2)
    g = (gid_ref[...] == lax.broadcasted_iota(jnp.int32, (RB, B), 1)).astype(f32)
    acc_ref[...] += lax.dot_general(
        g, h2, (((0,), (0,)), ((), ())), preferred_element_type=f32)

    @pl.when(i == GRID - 1)
    def _():
        hgs = acc_ref[...]
        cnt = hgs[:, 20:21]
        hg = hgs / jnp.maximum(cnt, 1.0)
        lane32 = lax.broadcasted_iota(jnp.int32, hg.shape, 1)
        hg = jnp.where(lane32 >= 20, 0.0, hg)
        d2 = d2_ref[...]
        z = (jnp.dot(hg, gwh_ref[...], preferred_element_type=f32)
             + jnp.dot(d2, gwd_ref[...], preferred_element_type=f32)
             + gb_ref[...])
        g2 = 1.0 / (1.0 + jnp.exp(-z))
        v2 = g2 * d2
        out1 = fb1_ref[...] + jnp.zeros((B, 128), f32)
        for ii in range(21):
            t = (jnp.dot(v2, wv_ref[ii], preferred_element_type=f32)
                 + wc_ref[ii:ii + 1, :])
            if ii == 20:
                out1 = out1 + t
            else:
                out1 = out1 + hg[:, ii:ii + 1] * t
        mu1 = jnp.mean(out1, axis=0, keepdims=True)
        var1 = jnp.mean((out1 - mu1) ** 2, axis=0, keepdims=True)
        h = jnp.maximum((out1 - mu1) / jnp.sqrt(var1 + 1e-5) * bn1g_ref[...]
                        + bn1b_ref[...], 0.0)
        o2 = jnp.dot(h, f2t_ref[...], preferred_element_type=f32) + f2b_ref[...]
        mu2 = jnp.mean(o2, axis=0, keepdims=True)
        var2 = jnp.mean((o2 - mu2) ** 2, axis=0, keepdims=True)
        o2 = jnp.maximum((o2 - mu2) / jnp.sqrt(var2 + 1e-5) * bn2g_ref[...]
                         + bn2b_ref[...], 0.0)
        o3 = jnp.sum(o2 * f3_ref[...], axis=1, keepdims=True) + f3b_ref[...]
        o_ref[...] = o3


def _tc_head(y2, a0, gids, b2p, d2, gwh, gwd, gb, wv, wc, fb1,
             bn1g, bn1b, f2t, f2b, bn2g, bn2b, f3, f3b):
    blk = lambda r, c: pl.BlockSpec((r, c), lambda i: (i, 0))
    cst = lambda *s: pl.BlockSpec(s, lambda i: tuple(0 for _ in s))
    return pl.pallas_call(
        _head_body,
        grid=(GRID,),
        in_specs=[
            blk(RB, D2), blk(RB, D2),
            pl.BlockSpec((RB, 1), lambda i: (i, 0)),
            cst(1, D2), cst(B, 200), cst(D2, 200), cst(200, 200), cst(1, 200),
            cst(21, 200, 128), cst(24, 128), cst(1, 128),
            cst(1, 128), cst(1, 128), cst(128, D2), cst(1, D2),
            cst(1, D2), cst(1, D2), cst(1, D2), cst(1, 1),
        ],
        out_specs=pl.BlockSpec((B, 1), lambda i: (0, 0)),
        out_shape=jax.ShapeDtypeStruct((B, 1), f32),
        scratch_shapes=[pltpu.VMEM((B, D2), f32)],
    )(y2, a0, gids, b2p, d2, gwh, gwd, gb, wv, wc, fb1,
      bn1g, bn1b, f2t, f2b, bn2g, bn2b, f3, f3b)


def kernel(x, edge_index, node_graph_ids, desc_2d, desc_3d,
           W1, b1, W2, b2, gate_W, gate_b,
           fc1_W, fc1_b, fc2_W, fc2_b, fc3_W, fc3_b,
           bn1_g, bn1_b, bn2_g, bn2_b):
    del desc_3d
    i32 = jnp.int32
    xp = jnp.pad(x, ((0, NP - N), (0, 0)))
    src = jnp.concatenate(
        [edge_index[0].astype(i32), jnp.full((EP - E,), N, i32)])
    dst = jnp.concatenate(
        [edge_index[1].astype(i32), N + jnp.arange(EP - E, dtype=i32) % (NP - N)])
    srcT = src.reshape(16, EP // 16)
    dstT1 = dst.reshape(16, EP // 16 // K1, K1)
    dstT2b = dst.reshape(16, EP // 16 // 64, 64)
    gids = jnp.concatenate(
        [node_graph_ids.astype(i32), jnp.full((NP - N,), B, i32)]
    ).reshape(NP, 1)

    w1t = jnp.pad(W1, ((0, D1 - 100), (0, 0))).T
    b1p = jnp.pad(b1, (0, D1 - 100)).reshape(1, D1)
    w2t = jnp.pad(W2, ((0, D2 - 20), (0, D1 - 100))).T
    b2p = jnp.pad(b2, (0, D2 - 20)).reshape(1, D2)
    gwh = jnp.pad(gate_W[:, :20].T, ((0, D2 - 20), (0, 0)))
    gwd = gate_W[:, 20:].T
    gb = gate_b.reshape(1, 200)
    wv3 = fc1_W.T.reshape(21, 201, 128)
    wv = wv3[:, :200, :]
    wc = jnp.pad(wv3[:, 200, :], ((0, 3), (0, 0)))
    fb1 = fc1_b.reshape(1, 128)
    f2t = fc2_W.T
    f2b = fc2_b.reshape(1, D2)
    f3 = fc3_W.reshape(1, 32)
    f3b = fc3_b.reshape(1, 1)
    bn1g = bn1_g.reshape(1, 128)
    bn1b = bn1_b.reshape(1, 128)
    bn2g = bn2_g.reshape(1, D2)
    bn2b = bn2_b.reshape(1, D2)

    z1 = jnp.zeros((STR, D1), f32)
    z2 = jnp.zeros((STR, D2), f32)

    y1 = _tc_y1(xp, w1t)
    agg1 = _make_sc_agg(D1, K1, 2, 1)(y1, srcT, dstT1, z1)
    y2 = _tc_mid(y1, agg1, w2t, b1p)
    agg2 = _make_sc_agg(D2, 64, 8, 1)(y2, srcT, dstT2b, z2)
    return _tc_head(y2, agg2, gids, b2p, desc_2d, gwh, gwd, gb, wv, wc,
                    fb1, bn1g, bn1b, f2t, f2b, bn2g, bn2b, f3, f3b)


# speedup vs baseline: 1.1391x; 1.1391x over previous
import functools

import jax
import jax.numpy as jnp
from jax import lax
from jax.experimental import pallas as pl
from jax.experimental.pallas import tpu as pltpu
from jax.experimental.pallas import tpu_sc as plsc

N = 10000
E = 160000
B = 64
NP = 10240
EP = 163840
NTILES = 32
EPT = EP // NTILES
K1 = 64
K2 = 128
STR = NP // 16
D1 = 112
D2 = 32
RB = 1024
GRID = NP // RB

f32 = jnp.float32


def _mm_body(x_ref, w_ref, o_ref):
    y = jnp.dot(x_ref[...], w_ref[...], preferred_element_type=f32)
    lane = lax.broadcasted_iota(jnp.int32, y.shape, 1)
    o_ref[...] = jnp.where(lane == 100, 1.0, y)


def _tc_y1(xp, w1t):
    return pl.pallas_call(
        _mm_body,
        grid=(GRID,),
        in_specs=[
            pl.BlockSpec((RB, 256), lambda i: (i, 0)),
            pl.BlockSpec((256, D1), lambda i: (0, 0)),
        ],
        out_specs=pl.BlockSpec((RB, D1), lambda i: (i, 0)),
        out_shape=jax.ShapeDtypeStruct((NP, D1), f32),
    )(xp, w1t)


@functools.lru_cache(maxsize=None)
def _make_sc_agg(Dh, KD, GRP):
    EPTc = EP // 16
    NCH = EPTc // KD
    NBODY = NCH // (2 * GRP)
    mesh = plsc.VectorSubcoreMesh(
        core_axis_name="c", subcore_axis_name="s", num_cores=2, num_subcores=16
    )

    @functools.partial(
        pl.kernel,
        out_type=jax.ShapeDtypeStruct((2 * NP, Dh), f32),
        mesh=mesh,
        compiler_params=pltpu.CompilerParams(use_tc_tiling_on_sc=False),
        scratch_types=[
            pltpu.VMEM((EPTc,), jnp.int32),
            pltpu.VMEM((NCH, KD), jnp.int32),
            pltpu.VMEM((2 * GRP, KD, Dh), f32),
            pltpu.VMEM_SHARED((NP, Dh), f32),
            pltpu.SemaphoreType.DMA,
            pltpu.SemaphoreType.DMA,
            pltpu.SemaphoreType.DMA,
            pltpu.SemaphoreType.DMA,
        ],
    )
    def sc_agg(y_hbm, src_hbm, dst_hbm, z_hbm, out_hbm,
               src_all, dst_all, rows, agg, gsa, gsb, ssa, ssb):
        cid = lax.axis_index("c")
        sid = lax.axis_index("s")
        wid = sid

        def fire_g(c0, b0, sem):
            for b in range(GRP):
                idx = src_all.at[pl.ds((c0 + b) * KD, KD)]
                pltpu.async_copy(y_hbm.at[idx], rows.at[b0 + b], sem)

        def drain_g(c0, b0, sem):
            for b in range(GRP):
                idx = src_all.at[pl.ds((c0 + b) * KD, KD)]
                pltpu.make_async_copy(y_hbm.at[idx], rows.at[b0 + b], sem).wait()

        def fire_s(c0, b0, sem):
            for b in range(GRP):
                pltpu.async_copy(rows.at[b0 + b], agg.at[dst_all.at[c0 + b]],
                                 sem, add=True)

        def drain_s(c0, b0, sem):
            for b in range(GRP):
                pltpu.make_async_copy(rows.at[b0 + b],
                                      agg.at[dst_all.at[c0 + b]], sem).wait()

        pltpu.sync_copy(src_hbm.at[cid, wid], src_all)
        pltpu.sync_copy(dst_hbm.at[wid], dst_all)
        pltpu.sync_copy(z_hbm, agg.at[pl.ds(sid * STR, STR)])
        plsc.subcore_barrier()

        fire_g(0, 0, gsa)

        def body(g, carry):
            cA = g * 2 * GRP
            cB = cA + GRP

            @pl.when(g > 0)
            def _():
                drain_s(cA - GRP, GRP, ssb)

            fire_g(cB, GRP, gsb)
            drain_g(cA, 0, gsa)
            fire_s(cA, 0, ssa)
            drain_g(cB, GRP, gsb)
            fire_s(cB, GRP, ssb)
            drain_s(cA, 0, ssa)

            @pl.when(g < NBODY - 1)
            def _():
                fire_g(cA + 2 * GRP, 0, gsa)

            return carry

        lax.fori_loop(0, NBODY, body, 0)
        drain_s((NBODY - 1) * 2 * GRP + GRP, GRP, ssb)
        plsc.subcore_barrier()
        row0 = cid * NP + sid * STR
        pltpu.sync_copy(agg.at[pl.ds(sid * STR, STR)], out_hbm.at[pl.ds(row0, STR)])

    return sc_agg


def _mid_body(y1_ref, a0_ref, a1_ref, w_ref, b_ref, o_ref):
    a = jnp.concatenate([a0_ref[...], a1_ref[...]], axis=1)
    deg = a[:, 100:101]
    mean = a / jnp.maximum(deg, 1.0)
    h1 = jnp.maximum(jnp.where(deg > 0, mean, y1_ref[...]) + b_ref[...], 0.0)
    y2 = jnp.dot(h1, w_ref[...], preferred_element_type=f32)
    lane = lax.broadcasted_iota(jnp.int32, y2.shape, 1)
    o_ref[...] = jnp.where(lane == 20, 1.0, jnp.where(lane == 21, deg, y2))


def _tc_mid(y1, a0, a1, w2t, b1p):
    return pl.pallas_call(
        _mid_body,
        grid=(GRID,),
        in_specs=[
            pl.BlockSpec((RB, D1), lambda i: (i, 0)),
            pl.BlockSpec((RB, D1 // 2), lambda i: (i, 0)),
            pl.BlockSpec((RB, D1 // 2), lambda i: (i, 0)),
            pl.BlockSpec((D1, D2), lambda i: (0, 0)),
            pl.BlockSpec((1, D1), lambda i: (0, 0)),
        ],
        out_specs=pl.BlockSpec((RB, D2), lambda i: (i, 0)),
        out_shape=jax.ShapeDtypeStruct((NP, D2), f32),
    )(y1, a0, a1, w2t, b1p)


def _head_body(y2_ref, a0_ref, a1_ref, gid_ref, b2_ref, d2_ref,
               gwh_ref, gwd_ref, gb_ref, wv_ref, wc_ref, fb1_ref,
               bn1g_ref, bn1b_ref, f2t_ref, f2b_ref, bn2g_ref, bn2b_ref,
               f3_ref, f3b_ref, o_ref, acc_ref):
    i = pl.program_id(0)

    @pl.when(i == 0)
    def _():
        acc_ref[...] = jnp.zeros((B, D2), f32)

    a = jnp.concatenate([a0_ref[...], a1_ref[...]], axis=1)
    y2 = y2_ref[...]
    deg = y2[:, 21:22]
    mean = a / jnp.maximum(deg, 1.0)
    h2 = jnp.maximum(jnp.where(deg > 0, mean, y2) + b2_ref[...], 0.0)
    lane = lax.broadcasted_iota(jnp.int32, h2.shape, 1)
    h2 = jnp.where(lane > 20, 0.0, h2)
    g = (gid_ref[...] == lax.broadcasted_iota(jnp.int32, (RB, B), 1)).astype(f32)
    acc_ref[...] += lax.dot_general(
        g, h2, (((0,), (0,)), ((), ())), preferred_element_type=f32)

    @pl.when(i == GRID - 1)
    def _():
        hgs = acc_ref[...]
        cnt = hgs[:, 20:21]
        hg = hgs / jnp.maximum(cnt, 1.0)
        lane32 = lax.broadcasted_iota(jnp.int32, hg.shape, 1)
        hg = jnp.where(lane32 >= 20, 0.0, hg)
        d2 = d2_ref[...]
        z = (jnp.dot(hg, gwh_ref[...], preferred_element_type=f32)
             + jnp.dot(d2, gwd_ref[...], preferred_element_type=f32)
             + gb_ref[...])
        g2 = 1.0 / (1.0 + jnp.exp(-z))
        v2 = g2 * d2
        out1 = fb1_ref[...] + jnp.zeros((B, 128), f32)
        for ii in range(21):
            t = (jnp.dot(v2, wv_ref[ii], preferred_element_type=f32)
                 + wc_ref[ii:ii + 1, :])
            if ii == 20:
                out1 = out1 + t
            else:
                out1 = out1 + hg[:, ii:ii + 1] * t
        mu1 = jnp.mean(out1, axis=0, keepdims=True)
        var1 = jnp.mean((out1 - mu1) ** 2, axis=0, keepdims=True)
        h = jnp.maximum((out1 - mu1) / jnp.sqrt(var1 + 1e-5) * bn1g_ref[...]
                        + bn1b_ref[...], 0.0)
        o2 = jnp.dot(h, f2t_ref[...], preferred_element_type=f32) + f2b_ref[...]
        mu2 = jnp.mean(o2, axis=0, keepdims=True)
        var2 = jnp.mean((o2 - mu2) ** 2, axis=0, keepdims=True)
        o2 = jnp.maximum((o2 - mu2) / jnp.sqrt(var2 + 1e-5) * bn2g_ref[...]
                         + bn2b_ref[...], 0.0)
        o3 = jnp.sum(o2 * f3_ref[...], axis=1, keepdims=True) + f3b_ref[...]
        o_ref[...] = o3


def _tc_head(y2, a0, a1, gids, b2p, d2, gwh, gwd, gb, wv, wc, fb1,
             bn1g, bn1b, f2t, f2b, bn2g, bn2b, f3, f3b):
    blk = lambda r, c: pl.BlockSpec((r, c), lambda i: (i, 0))
    cst = lambda *s: pl.BlockSpec(s, lambda i: tuple(0 for _ in s))
    return pl.pallas_call(
        _head_body,
        grid=(GRID,),
        in_specs=[
            blk(RB, D2), blk(RB, D2 // 2), blk(RB, D2 // 2),
            pl.BlockSpec((RB, 1), lambda i: (i, 0)),
            cst(1, D2), cst(B, 200), cst(D2, 200), cst(200, 200), cst(1, 200),
            cst(21, 200, 128), cst(24, 128), cst(1, 128),
            cst(1, 128), cst(1, 128), cst(128, D2), cst(1, D2),
            cst(1, D2), cst(1, D2), cst(1, D2), cst(1, 1),
        ],
        out_specs=pl.BlockSpec((B, 1), lambda i: (0, 0)),
        out_shape=jax.ShapeDtypeStruct((B, 1), f32),
        scratch_shapes=[pltpu.VMEM((B, D2), f32)],
    )(y2, a0, a1, gids, b2p, d2, gwh, gwd, gb, wv, wc, fb1,
      bn1g, bn1b, f2t, f2b, bn2g, bn2b, f3, f3b)


def kernel(x, edge_index, node_graph_ids, desc_2d, desc_3d,
           W1, b1, W2, b2, gate_W, gate_b,
           fc1_W, fc1_b, fc2_W, fc2_b, fc3_W, fc3_b,
           bn1_g, bn1_b, bn2_g, bn2_b):
    del desc_3d
    i32 = jnp.int32
    xp = jnp.pad(x, ((0, NP - N), (0, 0)))
    src = jnp.concatenate(
        [edge_index[0].astype(i32), jnp.full((EP - E,), N, i32)])
    dst = jnp.concatenate(
        [edge_index[1].astype(i32), N + jnp.arange(EP - E, dtype=i32) % (NP - N)])
    srcT2 = jnp.stack([2 * src, 2 * src + 1]).reshape(2, 16, EP // 16)
    dstT1 = dst.reshape(16, EP // 16 // K1, K1)
    dstT2 = dst.reshape(16, EP // 16 // K2, K2)
    gids = jnp.concatenate(
        [node_graph_ids.astype(i32), jnp.full((NP - N,), B, i32)]
    ).reshape(NP, 1)

    w1t = jnp.pad(W1, ((0, D1 - 100), (0, 0))).T
    b1p = jnp.pad(b1, (0, D1 - 100)).reshape(1, D1)
    w2t = jnp.pad(W2, ((0, D2 - 20), (0, D1 - 100))).T
    b2p = jnp.pad(b2, (0, D2 - 20)).reshape(1, D2)
    gwh = jnp.pad(gate_W[:, :20].T, ((0, D2 - 20), (0, 0)))
    gwd = gate_W[:, 20:].T
    gb = gate_b.reshape(1, 200)
    wv3 = fc1_W.T.reshape(21, 201, 128)
    wv = wv3[:, :200, :]
    wc = jnp.pad(wv3[:, 200, :], ((0, 3), (0, 0)))
    fb1 = fc1_b.reshape(1, 128)
    f2t = fc2_W.T
    f2b = fc2_b.reshape(1, D2)
    f3 = fc3_W.reshape(1, 32)
    f3b = fc3_b.reshape(1, 1)
    bn1g = bn1_g.reshape(1, 128)
    bn1b = bn1_b.reshape(1, 128)
    bn2g = bn2_g.reshape(1, D2)
    bn2b = bn2_b.reshape(1, D2)

    z1 = jnp.zeros((STR, D1 // 2), f32)
    z2 = jnp.zeros((STR, D2 // 2), f32)

    y1 = _tc_y1(xp, w1t)
    agg1 = _make_sc_agg(D1 // 2, K1, 4)(
        y1.reshape(2 * NP, D1 // 2), srcT2, dstT1, z1)
    y2 = _tc_mid(y1, agg1[:NP], agg1[NP:], w2t, b1p)
    agg2 = _make_sc_agg(D2 // 2, K2, 4)(
        y2.reshape(2 * NP, D2 // 2), srcT2, dstT2, z2)
    return _tc_head(y2, agg2[:NP], agg2[NP:], gids, b2p, desc_2d, gwh, gwd,
                    gb, wv, wc, fb1, bn1g, bn1b, f2t, f2b, bn2g, bn2b, f3, f3b)
